# baseline (speedup 1.0000x reference)
import functools

import jax
import jax.numpy as jnp
from jax.experimental import pallas as pl
from jax.experimental.pallas import tpu as pltpu
from jax.experimental.pallas import tpu_sc as plsc

ALIGN = 8
BT = 32
RBLK = 128


def _dot_t(a, b):
    return jax.lax.dot_general(
        a, b, (((1,), (1,)), ((), ())), preferred_element_type=jnp.float32
    )


def _dot(a, b):
    return jax.lax.dot_general(
        a, b, (((1,), (0,)), ((), ())), preferred_element_type=jnp.float32
    )


def _gelu(x):
    return x * 0.5 * (1.0 + jax.lax.erf(x * 0.7071067811865476))


def _front_kernel(pbuf, x_ref, wg_ref, bg_ref, ws1_ref, bs1_ref, ws2_ref,
                  bs2_ref, pos_ref, posg_ref, po_ref, cnt_ref, sh_ref):
    T = x_ref.shape[0]
    E = wg_ref.shape[0]
    x = x_ref[...]

    logits = _dot_t(x, wg_ref[...]) + bg_ref[...]
    m = jnp.max(logits, axis=1, keepdims=True)
    cols = jax.lax.broadcasted_iota(jnp.int32, logits.shape, 1)
    eid = jnp.min(jnp.where(logits == m, cols, E), axis=1)
    ohf = (cols == eid[:, None]).astype(jnp.float32)

    counts = jnp.sum(ohf, axis=0)
    pc = jnp.floor((counts + (ALIGN - 1)) / ALIGN) * ALIGN
    er = jax.lax.broadcasted_iota(jnp.int32, (E, E), 0)
    ec = jax.lax.broadcasted_iota(jnp.int32, (E, E), 1)
    po = jnp.sum(jnp.where(ec < er, pc[None, :], 0.0), axis=1)

    li = jax.lax.broadcasted_iota(jnp.int32, (RBLK, RBLK), 0)
    lj = jax.lax.broadcasted_iota(jnp.int32, (RBLK, RBLK), 1)
    lower = jnp.where(lj < li, 1.0, 0.0)
    prefix = jnp.zeros((1, E), jnp.float32)
    for b in range(T // RBLK):
        ohb = jax.lax.slice(ohf, (b * RBLK, 0), ((b + 1) * RBLK, E))
        excl = _dot(lower, ohb)
        slot = jnp.sum((excl + prefix + po[None, :]) * ohb, axis=1)
        pos_ref[0, pl.ds(b * RBLK, RBLK)] = slot.astype(jnp.int32)
        hi = jnp.sum(jax.lax.slice(ohb, (0, E // 2), (RBLK, E)), axis=1)
        posg_ref[0, pl.ds(b * RBLK, RBLK)] = (
            slot + pbuf * hi).astype(jnp.int32)
        prefix = prefix + jnp.sum(ohb, axis=0, keepdims=True)
    po_ref[...] = po[None, :].astype(jnp.int32)
    cnt_ref[...] = counts[None, :].astype(jnp.int32)

    h = _gelu(_dot_t(x, ws1_ref[...]) + bs1_ref[...])
    sh_ref[...] = _dot_t(h, ws2_ref[...]) + bs2_ref[...]


def _sc_mesh():
    return plsc.VectorSubcoreMesh(core_axis_name="c", subcore_axis_name="s")


def _sc_workers():
    info = plsc.get_sparse_core_info()
    return info.num_cores, info.num_subcores


def _sc_dispatch(xa, xb, idx, nrows_out):
    n, d = xa.shape
    nc, ns = _sc_workers()
    chunk = n // (nc * ns)
    out_t = jax.ShapeDtypeStruct((nrows_out, d), xa.dtype)

    @functools.partial(
        pl.kernel,
        mesh=_sc_mesh(),
        out_type=(out_t, out_t),
        scratch_types=[
            pltpu.VMEM((chunk,), jnp.int32),
            pltpu.VMEM((chunk, d), xa.dtype),
            pltpu.SemaphoreType.DMA,
        ],
    )
    def kern(a_hbm, b_hbm, i_hbm, oa_hbm, ob_hbm, idx_v, rows_v, sem):
        wid = jax.lax.axis_index("s") * nc + jax.lax.axis_index("c")
        base = wid * chunk
        pltpu.sync_copy(i_hbm.at[pl.ds(base, chunk)], idx_v)
        pltpu.sync_copy(a_hbm.at[pl.ds(base, chunk)], rows_v)
        pltpu.async_copy(rows_v, oa_hbm.at[idx_v], sem).wait()
        pltpu.sync_copy(b_hbm.at[pl.ds(base, chunk)], rows_v)
        pltpu.async_copy(rows_v, ob_hbm.at[idx_v], sem).wait()

    return kern(xa, xb, idx)


def _sc_gather_rows(src, idx):
    n = idx.shape[0]
    d = src.shape[1]
    nc, ns = _sc_workers()
    chunk = n // (nc * ns)

    @functools.partial(
        pl.kernel,
        mesh=_sc_mesh(),
        out_type=jax.ShapeDtypeStruct((n, d), src.dtype),
        scratch_types=[
            pltpu.VMEM((chunk,), jnp.int32),
            pltpu.VMEM((chunk, d), src.dtype),
            pltpu.SemaphoreType.DMA,
        ],
    )
    def kern(x_hbm, i_hbm, o_hbm, idx_v, rows_v, sem):
        wid = jax.lax.axis_index("s") * nc + jax.lax.axis_index("c")
        base = wid * chunk
        pltpu.sync_copy(i_hbm.at[pl.ds(base, chunk)], idx_v)
        pltpu.async_copy(x_hbm.at[idx_v], rows_v, sem).wait()
        pltpu.sync_copy(rows_v, o_hbm.at[pl.ds(base, chunk)])

    return kern(src, idx)


def _expert_kernel(ne_core, po_ref, cnt_ref, xs_ref, init_ref, w1_ref, b1_ref,
                   w2_ref, b2_ref, out_ref):
    e = pl.program_id(0) * ne_core + pl.program_id(1)
    start = po_ref[e]
    cnt = cnt_ref[e]
    nt = (cnt + BT - 1) // BT
    w1 = w1_ref[0]
    w2 = w2_ref[0]
    b1 = b1_ref[0]
    b2 = b2_ref[0]

    def body(t, _):
        base = pl.multiple_of(start + t * BT, ALIGN)
        xt = xs_ref[pl.ds(base, BT), :]
        h = _gelu(_dot_t(xt, w1) + b1)
        out_ref[0, pl.ds(base, BT), :] = (
            _dot_t(h, w2) + b2 + init_ref[pl.ds(base, BT), :])
        return 0

    jax.lax.fori_loop(0, nt, body, 0)


def kernel(x, Wg, bg, W1, b1, W2, b2, Ws1, bs1, Ws2, bs2):
    T, D = x.shape
    E, H = b1.shape
    PBUF = ((T + E * (ALIGN - 1) + BT + BT - 1) // BT) * BT

    pos2, posg2, po2, cnt2, shared = pl.pallas_call(
        functools.partial(_front_kernel, PBUF),
        out_shape=(
            jax.ShapeDtypeStruct((1, T), jnp.int32),
            jax.ShapeDtypeStruct((1, T), jnp.int32),
            jax.ShapeDtypeStruct((1, E), jnp.int32),
            jax.ShapeDtypeStruct((1, E), jnp.int32),
            jax.ShapeDtypeStruct((T, D), jnp.float32),
        ),
        in_specs=[
            pl.BlockSpec((T, D), lambda: (0, 0)),
            pl.BlockSpec((E, D), lambda: (0, 0)),
            pl.BlockSpec((1, E), lambda: (0, 0)),
            pl.BlockSpec((H, D), lambda: (0, 0)),
            pl.BlockSpec((1, H), lambda: (0, 0)),
            pl.BlockSpec((D, H), lambda: (0, 0)),
            pl.BlockSpec((1, D), lambda: (0, 0)),
        ],
        out_specs=(
            pl.BlockSpec((1, T), lambda: (0, 0)),
            pl.BlockSpec((1, T), lambda: (0, 0)),
            pl.BlockSpec((1, E), lambda: (0, 0)),
            pl.BlockSpec((1, E), lambda: (0, 0)),
            pl.BlockSpec((T, D), lambda: (0, 0)),
        ),
    )(x, Wg, bg.reshape(1, E), Ws1, bs1.reshape(1, H), Ws2, bs2.reshape(1, D))
    pos = pos2[0]
    posg = posg2[0]
    po = po2[0]
    counts = cnt2[0]

    xs, init = _sc_dispatch(x, shared, pos, PBUF)

    NEC = E // 2
    grid_spec = pltpu.PrefetchScalarGridSpec(
        num_scalar_prefetch=2,
        grid=(2, NEC),
        in_specs=[
            pl.BlockSpec((PBUF, D), lambda c, e, po_, c_: (0, 0)),
            pl.BlockSpec((PBUF, D), lambda c, e, po_, c_: (0, 0)),
            pl.BlockSpec((1, H, D), lambda c, e, po_, c_: (c * NEC + e, 0, 0)),
            pl.BlockSpec((1, 1, H), lambda c, e, po_, c_: (c * NEC + e, 0, 0)),
            pl.BlockSpec((1, D, H), lambda c, e, po_, c_: (c * NEC + e, 0, 0)),
            pl.BlockSpec((1, 1, D), lambda c, e, po_, c_: (c * NEC + e, 0, 0)),
        ],
        out_specs=pl.BlockSpec((1, PBUF, D), lambda c, e, po_, c_: (c, 0, 0)),
    )
    ys = pl.pallas_call(
        functools.partial(_expert_kernel, NEC),
        grid_spec=grid_spec,
        out_shape=jax.ShapeDtypeStruct((2, PBUF, D), jnp.float32),
        compiler_params=pltpu.CompilerParams(
            dimension_semantics=("parallel", "arbitrary")),
    )(po, counts, xs, init, W1, b1.reshape(E, 1, H), W2, b2.reshape(E, 1, D))

    return _sc_gather_rows(ys.reshape(2 * PBUF, D), posg)


# speedup vs baseline: 1.1042x; 1.1042x over previous
import functools

import jax
import jax.numpy as jnp
from jax.experimental import pallas as pl
from jax.experimental.pallas import tpu as pltpu
from jax.experimental.pallas import tpu_sc as plsc

ALIGN = 8
BT = 64
RBLK = 128


def _dot_t(a, b):
    return jax.lax.dot_general(
        a, b, (((1,), (1,)), ((), ())), preferred_element_type=jnp.float32
    )


def _dot(a, b):
    return jax.lax.dot_general(
        a, b, (((1,), (0,)), ((), ())), preferred_element_type=jnp.float32
    )


def _gelu(x):
    return x * 0.5 * (1.0 + jax.lax.erf(x * 0.7071067811865476))


def _front_kernel(pbuf, x_ref, wg_ref, bg_ref, ws1_ref, bs1_ref, ws2_ref,
                  bs2_ref, pos_ref, posg_ref, po_ref, cnt_ref, sh_ref):
    T = x_ref.shape[0]
    E = wg_ref.shape[0]
    x = x_ref[...]

    logits = _dot_t(x, wg_ref[...]) + bg_ref[...]
    m = jnp.max(logits, axis=1, keepdims=True)
    cols = jax.lax.broadcasted_iota(jnp.int32, logits.shape, 1)
    eid = jnp.min(jnp.where(logits == m, cols, E), axis=1)
    ohf = (cols == eid[:, None]).astype(jnp.float32)

    counts = jnp.sum(ohf, axis=0)
    pc = jnp.floor((counts + (ALIGN - 1)) / ALIGN) * ALIGN
    er = jax.lax.broadcasted_iota(jnp.int32, (E, E), 0)
    ec = jax.lax.broadcasted_iota(jnp.int32, (E, E), 1)
    po = jnp.sum(jnp.where(ec < er, pc[None, :], 0.0), axis=1)

    li = jax.lax.broadcasted_iota(jnp.int32, (RBLK, RBLK), 0)
    lj = jax.lax.broadcasted_iota(jnp.int32, (RBLK, RBLK), 1)
    lower = jnp.where(lj < li, 1.0, 0.0)
    prefix = jnp.zeros((1, E), jnp.float32)
    for b in range(T // RBLK):
        ohb = jax.lax.slice(ohf, (b * RBLK, 0), ((b + 1) * RBLK, E))
        excl = _dot(lower, ohb)
        slot = jnp.sum((excl + prefix + po[None, :]) * ohb, axis=1)
        pos_ref[0, pl.ds(b * RBLK, RBLK)] = slot.astype(jnp.int32)
        hi = jnp.sum(jax.lax.slice(ohb, (0, E // 2), (RBLK, E)), axis=1)
        posg_ref[0, pl.ds(b * RBLK, RBLK)] = (
            slot + pbuf * hi).astype(jnp.int32)
        prefix = prefix + jnp.sum(ohb, axis=0, keepdims=True)
    po_ref[...] = po[None, :].astype(jnp.int32)
    cnt_ref[...] = counts[None, :].astype(jnp.int32)

    h = _gelu(_dot_t(x, ws1_ref[...]) + bs1_ref[...])
    sh_ref[...] = _dot_t(h, ws2_ref[...]) + bs2_ref[...]


def _sc_mesh():
    return plsc.VectorSubcoreMesh(core_axis_name="c", subcore_axis_name="s")


def _sc_workers():
    info = plsc.get_sparse_core_info()
    return info.num_cores, info.num_subcores


def _sc_dispatch(xa, xb, idx, nrows_out):
    n, d = xa.shape
    nc, ns = _sc_workers()
    chunk = n // (nc * ns)
    out_t = jax.ShapeDtypeStruct((nrows_out, d), xa.dtype)

    @functools.partial(
        pl.kernel,
        mesh=_sc_mesh(),
        out_type=(out_t, out_t),
        scratch_types=[
            pltpu.VMEM((chunk,), jnp.int32),
            pltpu.VMEM((chunk, d), xa.dtype),
            pltpu.SemaphoreType.DMA,
        ],
    )
    def kern(a_hbm, b_hbm, i_hbm, oa_hbm, ob_hbm, idx_v, rows_v, sem):
        wid = jax.lax.axis_index("s") * nc + jax.lax.axis_index("c")
        base = wid * chunk
        pltpu.sync_copy(i_hbm.at[pl.ds(base, chunk)], idx_v)
        pltpu.sync_copy(a_hbm.at[pl.ds(base, chunk)], rows_v)
        pltpu.async_copy(rows_v, oa_hbm.at[idx_v], sem).wait()
        pltpu.sync_copy(b_hbm.at[pl.ds(base, chunk)], rows_v)
        pltpu.async_copy(rows_v, ob_hbm.at[idx_v], sem).wait()

    return kern(xa, xb, idx)


def _sc_gather_rows(src, idx):
    n = idx.shape[0]
    d = src.shape[1]
    nc, ns = _sc_workers()
    chunk = n // (nc * ns)

    @functools.partial(
        pl.kernel,
        mesh=_sc_mesh(),
        out_type=jax.ShapeDtypeStruct((n, d), src.dtype),
        scratch_types=[
            pltpu.VMEM((chunk,), jnp.int32),
            pltpu.VMEM((chunk, d), src.dtype),
            pltpu.SemaphoreType.DMA,
        ],
    )
    def kern(x_hbm, i_hbm, o_hbm, idx_v, rows_v, sem):
        wid = jax.lax.axis_index("s") * nc + jax.lax.axis_index("c")
        base = wid * chunk
        pltpu.sync_copy(i_hbm.at[pl.ds(base, chunk)], idx_v)
        pltpu.async_copy(x_hbm.at[idx_v], rows_v, sem).wait()
        pltpu.sync_copy(rows_v, o_hbm.at[pl.ds(base, chunk)])

    return kern(src, idx)


def _expert_kernel(po_ref, cnt_ref, xs_ref, init_ref, w1_ref, b1_ref,
                   w2_ref, b2_ref, out_ref):
    e = pl.program_id(0)
    start = po_ref[e]
    cnt = cnt_ref[e]
    nt = (cnt + BT - 1) // BT
    w1 = w1_ref[0]
    w2 = w2_ref[0]
    b1 = b1_ref[0]
    b2 = b2_ref[0]

    def body(t, _):
        base = pl.multiple_of(start + t * BT, ALIGN)
        xt = xs_ref[pl.ds(base, BT), :]
        h = _gelu(_dot_t(xt, w1) + b1)
        out_ref[pl.ds(base, BT), :] = (
            _dot_t(h, w2) + b2 + init_ref[pl.ds(base, BT), :])
        return 0

    jax.lax.fori_loop(0, nt, body, 0)


def kernel(x, Wg, bg, W1, b1, W2, b2, Ws1, bs1, Ws2, bs2):
    T, D = x.shape
    E, H = b1.shape
    PBUF = ((T + E * (ALIGN - 1) + BT + BT - 1) // BT) * BT

    pos2, posg2, po2, cnt2, shared = pl.pallas_call(
        functools.partial(_front_kernel, PBUF),
        out_shape=(
            jax.ShapeDtypeStruct((1, T), jnp.int32),
            jax.ShapeDtypeStruct((1, T), jnp.int32),
            jax.ShapeDtypeStruct((1, E), jnp.int32),
            jax.ShapeDtypeStruct((1, E), jnp.int32),
            jax.ShapeDtypeStruct((T, D), jnp.float32),
        ),
        in_specs=[
            pl.BlockSpec((T, D), lambda: (0, 0)),
            pl.BlockSpec((E, D), lambda: (0, 0)),
            pl.BlockSpec((1, E), lambda: (0, 0)),
            pl.BlockSpec((H, D), lambda: (0, 0)),
            pl.BlockSpec((1, H), lambda: (0, 0)),
            pl.BlockSpec((D, H), lambda: (0, 0)),
            pl.BlockSpec((1, D), lambda: (0, 0)),
        ],
        out_specs=(
            pl.BlockSpec((1, T), lambda: (0, 0)),
            pl.BlockSpec((1, T), lambda: (0, 0)),
            pl.BlockSpec((1, E), lambda: (0, 0)),
            pl.BlockSpec((1, E), lambda: (0, 0)),
            pl.BlockSpec((T, D), lambda: (0, 0)),
        ),
    )(x, Wg, bg.reshape(1, E), Ws1, bs1.reshape(1, H), Ws2, bs2.reshape(1, D))
    pos = pos2[0]
    posg = posg2[0]
    po = po2[0]
    counts = cnt2[0]

    xs, init = _sc_dispatch(x, shared, pos, PBUF)

    grid_spec = pltpu.PrefetchScalarGridSpec(
        num_scalar_prefetch=2,
        grid=(E,),
        in_specs=[
            pl.BlockSpec((PBUF, D), lambda e, po_, c_: (0, 0)),
            pl.BlockSpec((PBUF, D), lambda e, po_, c_: (0, 0)),
            pl.BlockSpec((1, H, D), lambda e, po_, c_: (e, 0, 0)),
            pl.BlockSpec((1, 1, H), lambda e, po_, c_: (e, 0, 0)),
            pl.BlockSpec((1, D, H), lambda e, po_, c_: (e, 0, 0)),
            pl.BlockSpec((1, 1, D), lambda e, po_, c_: (e, 0, 0)),
        ],
        out_specs=pl.BlockSpec((PBUF, D), lambda e, po_, c_: (0, 0)),
    )
    ys = pl.pallas_call(
        _expert_kernel,
        grid_spec=grid_spec,
        out_shape=jax.ShapeDtypeStruct((PBUF, D), jnp.float32),
    )(po, counts, xs, init, W1, b1.reshape(E, 1, H), W2, b2.reshape(E, 1, D))

    return _sc_gather_rows(ys, pos)


# speedup vs baseline: 1.2008x; 1.0875x over previous
import functools

import jax
import jax.numpy as jnp
from jax.experimental import pallas as pl
from jax.experimental.pallas import tpu as pltpu
from jax.experimental.pallas import tpu_sc as plsc

ALIGN = 8
BT = 64
RBLK = 128
EPP = 2


def _dot_t(a, b):
    return jax.lax.dot_general(
        a, b, (((1,), (1,)), ((), ())), preferred_element_type=jnp.float32
    )


def _dot(a, b):
    return jax.lax.dot_general(
        a, b, (((1,), (0,)), ((), ())), preferred_element_type=jnp.float32
    )


def _gelu(x):
    return x * 0.5 * (1.0 + jax.lax.erf(x * 0.7071067811865476))


def _front_kernel(pbuf, x_ref, wg_ref, bg_ref, ws1_ref, bs1_ref, ws2_ref,
                  bs2_ref, pos_ref, posg_ref, po_ref, cnt_ref, sh_ref):
    T = x_ref.shape[0]
    E = wg_ref.shape[0]
    x = x_ref[...]

    logits = _dot_t(x, wg_ref[...]) + bg_ref[...]
    m = jnp.max(logits, axis=1, keepdims=True)
    cols = jax.lax.broadcasted_iota(jnp.int32, logits.shape, 1)
    eid = jnp.min(jnp.where(logits == m, cols, E), axis=1)
    ohf = (cols == eid[:, None]).astype(jnp.float32)

    counts = jnp.sum(ohf, axis=0)
    pc = jnp.floor((counts + (ALIGN - 1)) / ALIGN) * ALIGN
    er = jax.lax.broadcasted_iota(jnp.int32, (E, E), 0)
    ec = jax.lax.broadcasted_iota(jnp.int32, (E, E), 1)
    po = jnp.sum(jnp.where(ec < er, pc[None, :], 0.0), axis=1)

    li = jax.lax.broadcasted_iota(jnp.int32, (RBLK, RBLK), 0)
    lj = jax.lax.broadcasted_iota(jnp.int32, (RBLK, RBLK), 1)
    lower = jnp.where(lj < li, 1.0, 0.0)
    prefix = jnp.zeros((1, E), jnp.float32)
    for b in range(T // RBLK):
        ohb = jax.lax.slice(ohf, (b * RBLK, 0), ((b + 1) * RBLK, E))
        excl = _dot(lower, ohb)
        slot = jnp.sum((excl + prefix + po[None, :]) * ohb, axis=1)
        pos_ref[0, pl.ds(b * RBLK, RBLK)] = slot.astype(jnp.int32)
        hi = jnp.sum(jax.lax.slice(ohb, (0, E // 2), (RBLK, E)), axis=1)
        posg_ref[0, pl.ds(b * RBLK, RBLK)] = (
            slot + pbuf * hi).astype(jnp.int32)
        prefix = prefix + jnp.sum(ohb, axis=0, keepdims=True)
    po_ref[...] = po[None, :].astype(jnp.int32)
    cnt_ref[...] = counts[None, :].astype(jnp.int32)

    h = _gelu(_dot_t(x, ws1_ref[...]) + bs1_ref[...])
    sh_ref[...] = _dot_t(h, ws2_ref[...]) + bs2_ref[...]


def _sc_mesh():
    return plsc.VectorSubcoreMesh(core_axis_name="c", subcore_axis_name="s")


def _sc_workers():
    info = plsc.get_sparse_core_info()
    return info.num_cores, info.num_subcores


def _sc_dispatch(xa, xb, idx, nrows_out):
    n, d = xa.shape
    nc, ns = _sc_workers()
    chunk = n // (nc * ns)
    out_t = jax.ShapeDtypeStruct((nrows_out, d), xa.dtype)

    @functools.partial(
        pl.kernel,
        mesh=_sc_mesh(),
        out_type=(out_t, out_t),
        scratch_types=[
            pltpu.VMEM((chunk,), jnp.int32),
            pltpu.VMEM((chunk, d), xa.dtype),
            pltpu.SemaphoreType.DMA,
        ],
    )
    def kern(a_hbm, b_hbm, i_hbm, oa_hbm, ob_hbm, idx_v, rows_v, sem):
        wid = jax.lax.axis_index("s") * nc + jax.lax.axis_index("c")
        base = wid * chunk
        pltpu.sync_copy(i_hbm.at[pl.ds(base, chunk)], idx_v)
        pltpu.sync_copy(a_hbm.at[pl.ds(base, chunk)], rows_v)
        pltpu.async_copy(rows_v, oa_hbm.at[idx_v], sem).wait()
        pltpu.sync_copy(b_hbm.at[pl.ds(base, chunk)], rows_v)
        pltpu.async_copy(rows_v, ob_hbm.at[idx_v], sem).wait()

    return kern(xa, xb, idx)


def _sc_gather_rows(src, idx):
    n = idx.shape[0]
    d = src.shape[1]
    nc, ns = _sc_workers()
    chunk = n // (nc * ns)

    @functools.partial(
        pl.kernel,
        mesh=_sc_mesh(),
        out_type=jax.ShapeDtypeStruct((n, d), src.dtype),
        scratch_types=[
            pltpu.VMEM((chunk,), jnp.int32),
            pltpu.VMEM((chunk, d), src.dtype),
            pltpu.SemaphoreType.DMA,
        ],
    )
    def kern(x_hbm, i_hbm, o_hbm, idx_v, rows_v, sem):
        wid = jax.lax.axis_index("s") * nc + jax.lax.axis_index("c")
        base = wid * chunk
        pltpu.sync_copy(i_hbm.at[pl.ds(base, chunk)], idx_v)
        pltpu.async_copy(x_hbm.at[idx_v], rows_v, sem).wait()
        pltpu.sync_copy(rows_v, o_hbm.at[pl.ds(base, chunk)])

    return kern(src, idx)


def _expert_kernel(po_ref, cnt_ref, xs_ref, init_ref, w1_ref, b1_ref,
                   w2_ref, b2_ref, out_ref):
    for s in range(EPP):
        e = pl.program_id(0) * EPP + s
        start = po_ref[e]
        cnt = cnt_ref[e]
        nt = (cnt + BT - 1) // BT
        w1 = w1_ref[s]
        w2 = w2_ref[s]
        b1 = b1_ref[s]
        b2 = b2_ref[s]

        def body(t, _, start=start, w1=w1, w2=w2, b1=b1, b2=b2):
            base = pl.multiple_of(start + t * BT, ALIGN)
            xt = xs_ref[pl.ds(base, BT), :]
            h = _gelu(_dot_t(xt, w1) + b1)
            out_ref[pl.ds(base, BT), :] = (
                _dot_t(h, w2) + b2 + init_ref[pl.ds(base, BT), :])
            return 0

        jax.lax.fori_loop(0, nt, body, 0)


def kernel(x, Wg, bg, W1, b1, W2, b2, Ws1, bs1, Ws2, bs2):
    T, D = x.shape
    E, H = b1.shape
    PBUF = ((T + E * (ALIGN - 1) + BT + BT - 1) // BT) * BT

    pos2, posg2, po2, cnt2, shared = pl.pallas_call(
        functools.partial(_front_kernel, PBUF),
        out_shape=(
            jax.ShapeDtypeStruct((1, T), jnp.int32),
            jax.ShapeDtypeStruct((1, T), jnp.int32),
            jax.ShapeDtypeStruct((1, E), jnp.int32),
            jax.ShapeDtypeStruct((1, E), jnp.int32),
            jax.ShapeDtypeStruct((T, D), jnp.float32),
        ),
        in_specs=[
            pl.BlockSpec((T, D), lambda: (0, 0)),
            pl.BlockSpec((E, D), lambda: (0, 0)),
            pl.BlockSpec((1, E), lambda: (0, 0)),
            pl.BlockSpec((H, D), lambda: (0, 0)),
            pl.BlockSpec((1, H), lambda: (0, 0)),
            pl.BlockSpec((D, H), lambda: (0, 0)),
            pl.BlockSpec((1, D), lambda: (0, 0)),
        ],
        out_specs=(
            pl.BlockSpec((1, T), lambda: (0, 0)),
            pl.BlockSpec((1, T), lambda: (0, 0)),
            pl.BlockSpec((1, E), lambda: (0, 0)),
            pl.BlockSpec((1, E), lambda: (0, 0)),
            pl.BlockSpec((T, D), lambda: (0, 0)),
        ),
    )(x, Wg, bg.reshape(1, E), Ws1, bs1.reshape(1, H), Ws2, bs2.reshape(1, D))
    pos = pos2[0]
    posg = posg2[0]
    po = po2[0]
    counts = cnt2[0]

    xs, init = _sc_dispatch(x, shared, pos, PBUF)

    grid_spec = pltpu.PrefetchScalarGridSpec(
        num_scalar_prefetch=2,
        grid=(E // EPP,),
        in_specs=[
            pl.BlockSpec((PBUF, D), lambda e, po_, c_: (0, 0)),
            pl.BlockSpec((PBUF, D), lambda e, po_, c_: (0, 0)),
            pl.BlockSpec((EPP, H, D), lambda e, po_, c_: (e, 0, 0)),
            pl.BlockSpec((EPP, 1, H), lambda e, po_, c_: (e, 0, 0)),
            pl.BlockSpec((EPP, D, H), lambda e, po_, c_: (e, 0, 0)),
            pl.BlockSpec((EPP, 1, D), lambda e, po_, c_: (e, 0, 0)),
        ],
        out_specs=pl.BlockSpec((PBUF, D), lambda e, po_, c_: (0, 0)),
    )
    ys = pl.pallas_call(
        _expert_kernel,
        grid_spec=grid_spec,
        out_shape=jax.ShapeDtypeStruct((PBUF, D), jnp.float32),
    )(po, counts, xs, init, W1, b1.reshape(E, 1, H), W2, b2.reshape(E, 1, D))

    return _sc_gather_rows(ys, pos)
